# baseline (speedup 1.0000x reference)
import functools

import jax
import jax.numpy as jnp
from jax import lax
from jax.experimental import pallas as pl
from jax.experimental.pallas import tpu as pltpu
from jax.experimental.pallas import tpu_sc as plsc

_N = 4096
_K = 32
_OUT = 64
_NEG = float(jnp.finfo(jnp.float32).min)
_HALF = _N // 2


def _dot(a, b):
    return lax.dot_general(a, b, (((1,), (0,)), ((), ())),
                           preferred_element_type=jnp.float32)


def _ua_body(x_ref, wu_ref, wa_ref, b_ref, u_ref, a_ref):
    xb = x_ref[...]
    u_ref[...] = _dot(xb, wu_ref[...])
    a_ref[...] = _dot(xb, wa_ref[...]) + b_ref[...]


def _ua(x, Wu, Wa, b2):
    C = x.shape[1]
    R = 1024
    return pl.pallas_call(
        _ua_body,
        grid=(_N // R,),
        in_specs=[
            pl.BlockSpec((R, C), lambda i: (i, 0)),
            pl.BlockSpec((C, _OUT), lambda i: (0, 0)),
            pl.BlockSpec((C, _OUT), lambda i: (0, 0)),
            pl.BlockSpec((1, _OUT), lambda i: (0, 0)),
        ],
        out_specs=[
            pl.BlockSpec((R, _OUT), lambda i: (i, 0)),
            pl.BlockSpec((R, _OUT), lambda i: (i, 0)),
        ],
        out_shape=[
            jax.ShapeDtypeStruct((_N, _OUT), jnp.float32),
            jax.ShapeDtypeStruct((_N, _OUT), jnp.float32),
        ],
    )(x, Wu, Wa, b2)


_TOPK_R = 256


_Q = _N // 4


def _topk_body(base_blk, x_ref, xT2_ref, idx_ref,
               v0_ref, v1_ref, v2_ref, v3_ref,
               j0_ref, j1_ref, j2_ref, j3_ref):
    R = _TOPK_R
    xb = x_ref[...]
    xt2 = xT2_ref[...]
    xy2 = _dot(xb, xt2)
    sq = 0.25 * jnp.sum(xt2 * xt2, axis=0, keepdims=True)
    iota = lax.broadcasted_iota(jnp.int32, (R, _N), 1)
    lanek = lax.broadcasted_iota(jnp.int32, (R, _K), 1)
    blk = pl.program_id(0) + base_blk
    diag = lax.broadcasted_iota(jnp.int32, (R, 1), 0) + blk * R
    pdiag = (diag & 3) * _Q + (diag >> 2)
    s = jnp.where(iota == pdiag, _NEG, xy2 - sq)
    a, b, c, d = (s[:, 0:_Q], s[:, _Q:2 * _Q],
                  s[:, 2 * _Q:3 * _Q], s[:, 3 * _Q:4 * _Q])
    iotaq = lax.broadcasted_iota(jnp.int32, (R, _Q), 1)
    ja = (iotaq * 4).astype(jnp.float32)
    jb, jc, jd = ja + 1.0, ja + 2.0, ja + 3.0

    def ce(va, jja, vb, jjb):
        p = va >= vb
        return (jnp.where(p, va, vb), jnp.where(p, jja, jjb),
                jnp.where(p, vb, va), jnp.where(p, jjb, jja))

    a, ja, b, jb = ce(a, ja, b, jb)
    c, jc, d, jd = ce(c, jc, d, jd)
    b, jb, c, jc = ce(b, jb, c, jc)
    a, ja, b, jb = ce(a, ja, b, jb)
    c, jc, d, jd = ce(c, jc, d, jd)
    b, jb, c, jc = ce(b, jb, c, jc)
    v0_ref[...], v1_ref[...], v2_ref[...], v3_ref[...] = a, b, c, d
    j0_ref[...], j1_ref[...], j2_ref[...], j3_ref[...] = ja, jb, jc, jd

    def step(r, idx_acc):
        v0 = v0_ref[...]
        j0 = j0_ref[...]
        v = jnp.max(v0, axis=1, keepdims=True)
        jf = jnp.min(jnp.where(v0 == v, j0, 1e9), axis=1, keepdims=True)
        hit = j0 == jf
        v0_ref[...] = jnp.where(hit, v1_ref[...], v0)
        j0_ref[...] = jnp.where(hit, j1_ref[...], j0)
        v1_ref[...] = jnp.where(hit, v2_ref[...], v1_ref[...])
        j1_ref[...] = jnp.where(hit, j2_ref[...], j1_ref[...])
        v2_ref[...] = jnp.where(hit, v3_ref[...], v2_ref[...])
        j2_ref[...] = jnp.where(hit, j3_ref[...], j2_ref[...])
        v3_ref[...] = jnp.where(hit, _NEG, v3_ref[...])
        return jnp.where(lanek == r, jf.astype(jnp.int32), idx_acc)

    idx0 = jnp.where(lanek == 0, diag, jnp.zeros((R, _K), jnp.int32))
    idx_ref[...] = lax.fori_loop(1, _K, step, idx0)


def _edge_topk_half(x, xT2p, half):
    C = x.shape[1]
    R = _TOPK_R
    G = _HALF // R
    base_blk = half * G
    return pl.pallas_call(
        functools.partial(_topk_body, base_blk),
        grid=(G,),
        in_specs=[
            pl.BlockSpec((R, C), lambda i: (i + base_blk, 0)),
            pl.BlockSpec((C, _N), lambda i: (0, 0)),
        ],
        out_specs=pl.BlockSpec((R, _K), lambda i: (i, 0)),
        out_shape=jax.ShapeDtypeStruct((_HALF, _K), jnp.int32),
        scratch_shapes=[pltpu.VMEM((R, _Q), jnp.float32)
                        for _ in range(8)],
    )(x, xT2p)


_NW = 32
_ROWS_W = _HALF // _NW
_CH = 4
_NCH = _ROWS_W // _CH


def _sc_gather_relu_max(u, idx_flat, a_half):
    mesh = plsc.VectorSubcoreMesh(core_axis_name="c", subcore_axis_name="s")

    @functools.partial(
        pl.kernel,
        out_type=jax.ShapeDtypeStruct((_HALF, _OUT), jnp.float32),
        mesh=mesh,
        compiler_params=pltpu.CompilerParams(use_tc_tiling_on_sc=False),
        scratch_types=[
            pltpu.VMEM((_CH * _K,), jnp.int32),
            pltpu.VMEM((_CH * _K,), jnp.int32),
            pltpu.VMEM((_CH * _K, _OUT), jnp.float32),
            pltpu.VMEM((_CH * _K, _OUT), jnp.float32),
            pltpu.VMEM((_ROWS_W, _OUT), jnp.float32),
            pltpu.VMEM((_ROWS_W, _OUT), jnp.float32),
            pltpu.SemaphoreType.DMA,
            pltpu.SemaphoreType.DMA,
            pltpu.SemaphoreType.DMA,
        ],
    )
    def k(u_hbm, idx_hbm, a_hbm, out_hbm, idx0, idx1, rows0, rows1,
          a_v, o_v, sem0, sem1, sema):
        wid = lax.axis_index("s") * 2 + lax.axis_index("c")
        base = wid * _ROWS_W

        def fire(ci, idxbuf, rowsbuf, sem):
            pltpu.sync_copy(
                idx_hbm.at[pl.ds((base + ci * _CH) * _K, _CH * _K)], idxbuf)
            pltpu.async_copy(u_hbm.at[idxbuf], rowsbuf, sem)

        def wait(rowsbuf, sem):
            pltpu.make_async_copy(u_hbm.at[pl.ds(0, _CH * _K), :],
                                  rowsbuf, sem).wait()

        def compute(ci, rowsbuf):
            def row_do(r, _):
                def jstep(j, accs):
                    return tuple(
                        jnp.maximum(acc, rowsbuf[r * _K + j, pl.ds(c * 16, 16)])
                        for c, acc in enumerate(accs))
                accs = tuple(rowsbuf[r * _K, pl.ds(c * 16, 16)]
                             for c in range(_OUT // 16))
                accs = lax.fori_loop(1, _K, jstep, accs)
                orow = ci * _CH + r
                for c in range(_OUT // 16):
                    o_v[orow, pl.ds(c * 16, 16)] = jnp.maximum(
                        a_v[orow, pl.ds(c * 16, 16)] + accs[c], 0.0)
                return 0
            lax.fori_loop(0, _CH, row_do, 0)

        pltpu.async_copy(a_hbm.at[pl.ds(base, _ROWS_W), :], a_v, sema)
        fire(0, idx0, rows0, sem0)
        pltpu.make_async_copy(a_hbm.at[pl.ds(base, _ROWS_W), :],
                              a_v, sema).wait()

        def body(g, carry):
            ci0 = 2 * g
            fire(ci0 + 1, idx1, rows1, sem1)
            wait(rows0, sem0)
            compute(ci0, rows0)

            @pl.when(ci0 + 2 < _NCH)
            def _():
                fire(ci0 + 2, idx0, rows0, sem0)

            wait(rows1, sem1)
            compute(ci0 + 1, rows1)
            return carry

        lax.fori_loop(0, _NCH // 2, body, 0)
        pltpu.sync_copy(o_v, out_hbm.at[pl.ds(base, _ROWS_W), :])

    return k(u, idx_flat, a_half)


_HEAD_R = 512


def _head_body(x1_ref, x2_ref, x3_ref, wl_ref, bl_ref, wm1_ref, bm1_ref,
               wm2_ref, bm2_ref, wh_ref, bh_ref, o_ref):
    wl = wl_ref[...]
    h = (_dot(x1_ref[...], wl[0:_OUT]) + _dot(x2_ref[...], wl[_OUT:2 * _OUT])
         + _dot(x3_ref[...], wl[2 * _OUT:3 * _OUT]) + bl_ref[...])
    h = jnp.maximum(h, 0.0)
    h = jnp.maximum(_dot(h, wm1_ref[...]) + bm1_ref[...], 0.0)
    h = jnp.maximum(_dot(h, wm2_ref[...]) + bm2_ref[...], 0.0)
    o = _dot(h, wh_ref[...]) + bh_ref[...]
    shifted = o - jnp.max(o, axis=1, keepdims=True)
    o_ref[...] = shifted - jnp.log(
        jnp.sum(jnp.exp(shifted), axis=1, keepdims=True))


def _head(x1, x2, x3, Wl, bl, Wm1, bm1, Wm2, bm2, Wh, bh):
    R = _HEAD_R
    G = _N // R
    ncls = Wh.shape[1]
    full = lambda shp: pl.BlockSpec(shp, lambda i: tuple(0 for _ in shp))
    row = lambda shp: pl.BlockSpec(shp, lambda i: (i,) + (0,) * (len(shp) - 1))
    return pl.pallas_call(
        _head_body,
        grid=(G,),
        in_specs=[
            row((R, _OUT)), row((R, _OUT)), row((R, _OUT)),
            full(Wl.shape), full((1, bl.shape[0])),
            full(Wm1.shape), full((1, bm1.shape[0])),
            full(Wm2.shape), full((1, bm2.shape[0])),
            full(Wh.shape), full((1, bh.shape[0])),
        ],
        out_specs=row((R, ncls)),
        out_shape=jax.ShapeDtypeStruct((_N, ncls), jnp.float32),
    )(x1, x2, x3, Wl, bl.reshape(1, -1), Wm1, bm1.reshape(1, -1),
      Wm2, bm2.reshape(1, -1), Wh, bh.reshape(1, -1))


def _layer(xin, W, b):
    C = xin.shape[1]
    Wa = W[:C] - W[C:]
    Wu = W[C:]
    if C < 8:
        pad = 8 - C
        xin = jnp.concatenate([xin, jnp.zeros((_N, pad), xin.dtype)], axis=1)
        Wa = jnp.concatenate([Wa, jnp.zeros((pad, _OUT), Wa.dtype)], axis=0)
        Wu = jnp.concatenate([Wu, jnp.zeros((pad, _OUT), Wu.dtype)], axis=0)
    u, a = _ua(xin, Wu, Wa, b.reshape(1, _OUT))
    x2p = (xin + xin).reshape(_Q, 4, -1).transpose(1, 0, 2).reshape(_N, -1)
    xT2 = x2p.T
    idxA = _edge_topk_half(xin, xT2, 0)
    idxB = _edge_topk_half(xin, xT2, 1)
    xnA = _sc_gather_relu_max(u, idxA.reshape(_HALF * _K), a[:_HALF])
    xnB = _sc_gather_relu_max(u, idxB.reshape(_HALF * _K), a[_HALF:])
    return jnp.concatenate([xnA, xnB], axis=0)


def kernel(x, W1, b1, W2, b2, W3, b3, Wl, bl, Wm1, bm1, Wm2, bm2, Wh, bh):
    x1 = _layer(x, W1, b1)
    x2 = _layer(x1, W2, b2)
    x3 = _layer(x2, W3, b3)
    return _head(x1, x2, x3, Wl, bl, Wm1, bm1, Wm2, bm2, Wh, bh)


# speedup vs baseline: 1.2020x; 1.2020x over previous
import functools

import jax
import jax.numpy as jnp
from jax import lax
from jax.experimental import pallas as pl
from jax.experimental.pallas import tpu as pltpu
from jax.experimental.pallas import tpu_sc as plsc

_N = 4096
_K = 32
_OUT = 64
_NEG = float(jnp.finfo(jnp.float32).min)
_HALF = _N // 2


def _dot(a, b):
    return lax.dot_general(a, b, (((1,), (0,)), ((), ())),
                           preferred_element_type=jnp.float32)


def _ua_body(x_ref, wu_ref, wa_ref, b_ref, u_ref, a_ref):
    xb = x_ref[...]
    u_ref[...] = _dot(xb, wu_ref[...])
    a_ref[...] = _dot(xb, wa_ref[...]) + b_ref[...]


def _ua(x, Wu, Wa, b2):
    C = x.shape[1]
    R = 1024
    return pl.pallas_call(
        _ua_body,
        grid=(_N // R,),
        in_specs=[
            pl.BlockSpec((R, C), lambda i: (i, 0)),
            pl.BlockSpec((C, _OUT), lambda i: (0, 0)),
            pl.BlockSpec((C, _OUT), lambda i: (0, 0)),
            pl.BlockSpec((1, _OUT), lambda i: (0, 0)),
        ],
        out_specs=[
            pl.BlockSpec((R, _OUT), lambda i: (i, 0)),
            pl.BlockSpec((R, _OUT), lambda i: (i, 0)),
        ],
        out_shape=[
            jax.ShapeDtypeStruct((_N, _OUT), jnp.float32),
            jax.ShapeDtypeStruct((_N, _OUT), jnp.float32),
        ],
    )(x, Wu, Wa, b2)


_TOPK_R = 256


def _topk_body(base_blk, x_ref, xT2_ref, idx_ref, s_ref):
    R = _TOPK_R
    xb = x_ref[...]
    xt2 = xT2_ref[...]
    xy2 = _dot(xb, xt2)
    sq = 0.25 * jnp.sum(xt2 * xt2, axis=0, keepdims=True)
    iota = lax.broadcasted_iota(jnp.int32, (R, _N), 1)
    iotaf = iota.astype(jnp.float32)
    lanek = lax.broadcasted_iota(jnp.int32, (R, _K), 1)
    blk = pl.program_id(0) + base_blk
    diag = lax.broadcasted_iota(jnp.int32, (R, 1), 0) + blk * R
    s_ref[...] = jnp.where(iota == diag, _NEG, xy2 - sq)

    def step(r, idx_acc):
        sw = s_ref[...]
        v = jnp.max(sw, axis=1, keepdims=True)
        jf = jnp.min(jnp.where(sw == v, iotaf, 1e9), axis=1, keepdims=True)
        jstar = jf.astype(jnp.int32)
        s_ref[...] = jnp.where(iota == jstar, _NEG, sw)
        return jnp.where(lanek == r, jstar, idx_acc)

    idx0 = jnp.where(lanek == 0, diag, jnp.zeros((R, _K), jnp.int32))
    idx_ref[...] = lax.fori_loop(1, _K, step, idx0)


def _edge_topk_half(x, xT2, half):
    C = x.shape[1]
    R = _TOPK_R
    G = _HALF // R
    base_blk = half * G
    return pl.pallas_call(
        functools.partial(_topk_body, base_blk),
        grid=(G,),
        in_specs=[
            pl.BlockSpec((R, C), lambda i: (i + base_blk, 0)),
            pl.BlockSpec((C, _N), lambda i: (0, 0)),
        ],
        out_specs=pl.BlockSpec((R, _K), lambda i: (i, 0)),
        out_shape=jax.ShapeDtypeStruct((_HALF, _K), jnp.int32),
        scratch_shapes=[pltpu.VMEM((R, _N), jnp.float32)],
    )(x, xT2)


_NW = 32
_ROWS_W = _HALF // _NW
_CH = 4
_NCH = _ROWS_W // _CH


def _sc_gather_relu_max(u, idx_flat, a_half):
    mesh = plsc.VectorSubcoreMesh(core_axis_name="c", subcore_axis_name="s")

    @functools.partial(
        pl.kernel,
        out_type=jax.ShapeDtypeStruct((_HALF, _OUT), jnp.float32),
        mesh=mesh,
        compiler_params=pltpu.CompilerParams(use_tc_tiling_on_sc=False),
        scratch_types=[
            pltpu.VMEM((_CH * _K,), jnp.int32),
            pltpu.VMEM((_CH * _K,), jnp.int32),
            pltpu.VMEM((_CH * _K, _OUT), jnp.float32),
            pltpu.VMEM((_CH * _K, _OUT), jnp.float32),
            pltpu.VMEM((_ROWS_W, _OUT), jnp.float32),
            pltpu.VMEM((_ROWS_W, _OUT), jnp.float32),
            pltpu.SemaphoreType.DMA,
            pltpu.SemaphoreType.DMA,
            pltpu.SemaphoreType.DMA,
        ],
    )
    def k(u_hbm, idx_hbm, a_hbm, out_hbm, idx0, idx1, rows0, rows1,
          a_v, o_v, sem0, sem1, sema):
        wid = lax.axis_index("s") * 2 + lax.axis_index("c")
        base = wid * _ROWS_W

        def fire(ci, idxbuf, rowsbuf, sem):
            pltpu.sync_copy(
                idx_hbm.at[pl.ds((base + ci * _CH) * _K, _CH * _K)], idxbuf)
            pltpu.async_copy(u_hbm.at[idxbuf], rowsbuf, sem)

        def wait(rowsbuf, sem):
            pltpu.make_async_copy(u_hbm.at[pl.ds(0, _CH * _K), :],
                                  rowsbuf, sem).wait()

        def compute(ci, rowsbuf):
            def row_do(r, _):
                def jstep(j, accs):
                    return tuple(
                        jnp.maximum(acc, rowsbuf[r * _K + j, pl.ds(c * 16, 16)])
                        for c, acc in enumerate(accs))
                accs = tuple(rowsbuf[r * _K, pl.ds(c * 16, 16)]
                             for c in range(_OUT // 16))
                accs = lax.fori_loop(1, _K, jstep, accs)
                orow = ci * _CH + r
                for c in range(_OUT // 16):
                    o_v[orow, pl.ds(c * 16, 16)] = jnp.maximum(
                        a_v[orow, pl.ds(c * 16, 16)] + accs[c], 0.0)
                return 0
            lax.fori_loop(0, _CH, row_do, 0)

        pltpu.async_copy(a_hbm.at[pl.ds(base, _ROWS_W), :], a_v, sema)
        fire(0, idx0, rows0, sem0)
        pltpu.make_async_copy(a_hbm.at[pl.ds(base, _ROWS_W), :],
                              a_v, sema).wait()

        def body(g, carry):
            ci0 = 2 * g
            fire(ci0 + 1, idx1, rows1, sem1)
            wait(rows0, sem0)
            compute(ci0, rows0)

            @pl.when(ci0 + 2 < _NCH)
            def _():
                fire(ci0 + 2, idx0, rows0, sem0)

            wait(rows1, sem1)
            compute(ci0 + 1, rows1)
            return carry

        lax.fori_loop(0, _NCH // 2, body, 0)
        pltpu.sync_copy(o_v, out_hbm.at[pl.ds(base, _ROWS_W), :])

    return k(u, idx_flat, a_half)


_HEAD_R = 512


def _head_body(x1_ref, x2_ref, x3_ref, wl_ref, bl_ref, wm1_ref, bm1_ref,
               wm2_ref, bm2_ref, wh_ref, bh_ref, o_ref):
    wl = wl_ref[...]
    h = (_dot(x1_ref[...], wl[0:_OUT]) + _dot(x2_ref[...], wl[_OUT:2 * _OUT])
         + _dot(x3_ref[...], wl[2 * _OUT:3 * _OUT]) + bl_ref[...])
    h = jnp.maximum(h, 0.0)
    h = jnp.maximum(_dot(h, wm1_ref[...]) + bm1_ref[...], 0.0)
    h = jnp.maximum(_dot(h, wm2_ref[...]) + bm2_ref[...], 0.0)
    o = _dot(h, wh_ref[...]) + bh_ref[...]
    shifted = o - jnp.max(o, axis=1, keepdims=True)
    o_ref[...] = shifted - jnp.log(
        jnp.sum(jnp.exp(shifted), axis=1, keepdims=True))


def _head(x1, x2, x3, Wl, bl, Wm1, bm1, Wm2, bm2, Wh, bh):
    R = _HEAD_R
    G = _N // R
    ncls = Wh.shape[1]
    full = lambda shp: pl.BlockSpec(shp, lambda i: tuple(0 for _ in shp))
    row = lambda shp: pl.BlockSpec(shp, lambda i: (i,) + (0,) * (len(shp) - 1))
    return pl.pallas_call(
        _head_body,
        grid=(G,),
        in_specs=[
            row((R, _OUT)), row((R, _OUT)), row((R, _OUT)),
            full(Wl.shape), full((1, bl.shape[0])),
            full(Wm1.shape), full((1, bm1.shape[0])),
            full(Wm2.shape), full((1, bm2.shape[0])),
            full(Wh.shape), full((1, bh.shape[0])),
        ],
        out_specs=row((R, ncls)),
        out_shape=jax.ShapeDtypeStruct((_N, ncls), jnp.float32),
    )(x1, x2, x3, Wl, bl.reshape(1, -1), Wm1, bm1.reshape(1, -1),
      Wm2, bm2.reshape(1, -1), Wh, bh.reshape(1, -1))


def _layer(xin, W, b):
    C = xin.shape[1]
    Wa = W[:C] - W[C:]
    Wu = W[C:]
    if C < 8:
        pad = 8 - C
        xin = jnp.concatenate([xin, jnp.zeros((_N, pad), xin.dtype)], axis=1)
        Wa = jnp.concatenate([Wa, jnp.zeros((pad, _OUT), Wa.dtype)], axis=0)
        Wu = jnp.concatenate([Wu, jnp.zeros((pad, _OUT), Wu.dtype)], axis=0)
    u, a = _ua(xin, Wu, Wa, b.reshape(1, _OUT))
    xT2 = (xin + xin).T
    idxA = _edge_topk_half(xin, xT2, 0)
    idxB = _edge_topk_half(xin, xT2, 1)
    xnA = _sc_gather_relu_max(u, idxA.reshape(_HALF * _K), a[:_HALF])
    xnB = _sc_gather_relu_max(u, idxB.reshape(_HALF * _K), a[_HALF:])
    return jnp.concatenate([xnA, xnB], axis=0)


def kernel(x, W1, b1, W2, b2, W3, b3, Wl, bl, Wm1, bm1, Wm2, bm2, Wh, bh):
    x1 = _layer(x, W1, b1)
    x2 = _layer(x1, W2, b2)
    x3 = _layer(x2, W3, b3)
    return _head(x1, x2, x3, Wl, bl, Wm1, bm1, Wm2, bm2, Wh, bh)


# speedup vs baseline: 1.2033x; 1.0010x over previous
import functools

import jax
import jax.numpy as jnp
from jax import lax
from jax.experimental import pallas as pl
from jax.experimental.pallas import tpu as pltpu
from jax.experimental.pallas import tpu_sc as plsc

_N = 4096
_K = 32
_OUT = 64
_NEG = float(jnp.finfo(jnp.float32).min)
_HALF = _N // 2


def _dot(a, b):
    return lax.dot_general(a, b, (((1,), (0,)), ((), ())),
                           preferred_element_type=jnp.float32)


def _ua_body(x_ref, wu_ref, wa_ref, b_ref, u_ref, a_ref):
    xb = x_ref[...]
    u_ref[...] = _dot(xb, wu_ref[...])
    a_ref[...] = _dot(xb, wa_ref[...]) + b_ref[...]


def _ua(x, Wu, Wa, b2):
    C = x.shape[1]
    R = 1024
    return pl.pallas_call(
        _ua_body,
        grid=(_N // R,),
        in_specs=[
            pl.BlockSpec((R, C), lambda i: (i, 0)),
            pl.BlockSpec((C, _OUT), lambda i: (0, 0)),
            pl.BlockSpec((C, _OUT), lambda i: (0, 0)),
            pl.BlockSpec((1, _OUT), lambda i: (0, 0)),
        ],
        out_specs=[
            pl.BlockSpec((R, _OUT), lambda i: (i, 0)),
            pl.BlockSpec((R, _OUT), lambda i: (i, 0)),
        ],
        out_shape=[
            jax.ShapeDtypeStruct((_N, _OUT), jnp.float32),
            jax.ShapeDtypeStruct((_N, _OUT), jnp.float32),
        ],
    )(x, Wu, Wa, b2)


_TOPK_R = 256


def _topk_body(base_blk, x_ref, xT2_ref, idx_ref, s_ref):
    R = _TOPK_R
    xb = x_ref[...]
    xt2 = xT2_ref[...]
    xy2 = _dot(xb, xt2)
    sq = 0.25 * jnp.sum(xt2 * xt2, axis=0, keepdims=True)
    iota = lax.broadcasted_iota(jnp.int32, (R, _N), 1)
    iotaf = iota.astype(jnp.float32)
    lanek = lax.broadcasted_iota(jnp.int32, (R, _K), 1)
    blk = pl.program_id(0) + base_blk
    diag = lax.broadcasted_iota(jnp.int32, (R, 1), 0) + blk * R
    s_ref[...] = jnp.where(iota == diag, _NEG, xy2 - sq)

    def step(r, idx_acc):
        sw = s_ref[...]
        v = jnp.max(sw, axis=1, keepdims=True)
        jf = jnp.min(jnp.where(sw == v, iotaf, 1e9), axis=1, keepdims=True)
        jstar = jf.astype(jnp.int32)
        s_ref[...] = jnp.where(iota == jstar, _NEG, sw)
        return jnp.where(lanek == r, jstar, idx_acc)

    idx0 = jnp.where(lanek == 0, diag, jnp.zeros((R, _K), jnp.int32))
    idx_ref[...] = lax.fori_loop(1, _K, step, idx0)


def _edge_topk_half(x, xT2, half):
    C = x.shape[1]
    R = _TOPK_R
    G = _HALF // R
    base_blk = half * G
    return pl.pallas_call(
        functools.partial(_topk_body, base_blk),
        grid=(G,),
        in_specs=[
            pl.BlockSpec((R, C), lambda i: (i + base_blk, 0)),
            pl.BlockSpec((C, _N), lambda i: (0, 0)),
        ],
        out_specs=pl.BlockSpec((R, _K), lambda i: (i, 0)),
        out_shape=jax.ShapeDtypeStruct((_HALF, _K), jnp.int32),
        scratch_shapes=[pltpu.VMEM((R, _N), jnp.float32)],
    )(x, xT2)


_NW = 32
_ROWS_W = _HALF // _NW
_CH = 4
_NCH = _ROWS_W // _CH


def _sc_gather_relu_max(u, idx_flat, a_half):
    mesh = plsc.VectorSubcoreMesh(core_axis_name="c", subcore_axis_name="s")

    @functools.partial(
        pl.kernel,
        out_type=jax.ShapeDtypeStruct((_HALF, _OUT), jnp.float32),
        mesh=mesh,
        compiler_params=pltpu.CompilerParams(use_tc_tiling_on_sc=False),
        scratch_types=[
            pltpu.VMEM((_CH * _K,), jnp.int32),
            pltpu.VMEM((_CH * _K,), jnp.int32),
            pltpu.VMEM((_CH * _K, _OUT), jnp.float32),
            pltpu.VMEM((_CH * _K, _OUT), jnp.float32),
            pltpu.VMEM((_ROWS_W, _OUT), jnp.float32),
            pltpu.VMEM((_ROWS_W, _OUT), jnp.float32),
            pltpu.SemaphoreType.DMA,
            pltpu.SemaphoreType.DMA,
            pltpu.SemaphoreType.DMA,
        ],
    )
    def k(u_hbm, idx_hbm, a_hbm, out_hbm, idx0, idx1, rows0, rows1,
          a_v, o_v, sem0, sem1, sema):
        wid = lax.axis_index("s") * 2 + lax.axis_index("c")
        base = wid * _ROWS_W

        def fire(ci, idxbuf, rowsbuf, sem):
            pltpu.sync_copy(
                idx_hbm.at[pl.ds((base + ci * _CH) * _K, _CH * _K)], idxbuf)
            pltpu.async_copy(u_hbm.at[idxbuf], rowsbuf, sem)

        def wait(rowsbuf, sem):
            pltpu.make_async_copy(u_hbm.at[pl.ds(0, _CH * _K), :],
                                  rowsbuf, sem).wait()

        def compute(ci, rowsbuf):
            def row_do(r, _):
                def jstep(j, accs):
                    return tuple(
                        jnp.maximum(acc, rowsbuf[r * _K + j, pl.ds(c * 16, 16)])
                        for c, acc in enumerate(accs))
                accs = tuple(rowsbuf[r * _K, pl.ds(c * 16, 16)]
                             for c in range(_OUT // 16))
                accs = lax.fori_loop(1, _K, jstep, accs)
                orow = ci * _CH + r
                for c in range(_OUT // 16):
                    o_v[orow, pl.ds(c * 16, 16)] = jnp.maximum(
                        a_v[orow, pl.ds(c * 16, 16)] + accs[c], 0.0)
                return 0
            lax.fori_loop(0, _CH, row_do, 0)

        pltpu.async_copy(a_hbm.at[pl.ds(base, _ROWS_W), :], a_v, sema)
        fire(0, idx0, rows0, sem0)
        pltpu.make_async_copy(a_hbm.at[pl.ds(base, _ROWS_W), :],
                              a_v, sema).wait()

        def body(g, carry):
            ci0 = 2 * g
            fire(ci0 + 1, idx1, rows1, sem1)
            wait(rows0, sem0)
            compute(ci0, rows0)

            @pl.when(ci0 + 2 < _NCH)
            def _():
                fire(ci0 + 2, idx0, rows0, sem0)

            wait(rows1, sem1)
            compute(ci0 + 1, rows1)
            return carry

        lax.fori_loop(0, _NCH // 2, body, 0)
        pltpu.sync_copy(o_v, out_hbm.at[pl.ds(base, _ROWS_W), :])

    return k(u, idx_flat, a_half)


_HEAD_R = 1024


def _head_body(x1_ref, x2_ref, x3_ref, wl_ref, bl_ref, wm1_ref, bm1_ref,
               wm2_ref, bm2_ref, wh_ref, bh_ref, o_ref):
    wl = wl_ref[...]
    h = (_dot(x1_ref[...], wl[0:_OUT]) + _dot(x2_ref[...], wl[_OUT:2 * _OUT])
         + _dot(x3_ref[...], wl[2 * _OUT:3 * _OUT]) + bl_ref[...])
    h = jnp.maximum(h, 0.0)
    h = jnp.maximum(_dot(h, wm1_ref[...]) + bm1_ref[...], 0.0)
    h = jnp.maximum(_dot(h, wm2_ref[...]) + bm2_ref[...], 0.0)
    o = _dot(h, wh_ref[...]) + bh_ref[...]
    shifted = o - jnp.max(o, axis=1, keepdims=True)
    o_ref[...] = shifted - jnp.log(
        jnp.sum(jnp.exp(shifted), axis=1, keepdims=True))


def _head(x1, x2, x3, Wl, bl, Wm1, bm1, Wm2, bm2, Wh, bh):
    R = _HEAD_R
    G = _N // R
    ncls = Wh.shape[1]
    full = lambda shp: pl.BlockSpec(shp, lambda i: tuple(0 for _ in shp))
    row = lambda shp: pl.BlockSpec(shp, lambda i: (i,) + (0,) * (len(shp) - 1))
    return pl.pallas_call(
        _head_body,
        grid=(G,),
        in_specs=[
            row((R, _OUT)), row((R, _OUT)), row((R, _OUT)),
            full(Wl.shape), full((1, bl.shape[0])),
            full(Wm1.shape), full((1, bm1.shape[0])),
            full(Wm2.shape), full((1, bm2.shape[0])),
            full(Wh.shape), full((1, bh.shape[0])),
        ],
        out_specs=row((R, ncls)),
        out_shape=jax.ShapeDtypeStruct((_N, ncls), jnp.float32),
    )(x1, x2, x3, Wl, bl.reshape(1, -1), Wm1, bm1.reshape(1, -1),
      Wm2, bm2.reshape(1, -1), Wh, bh.reshape(1, -1))


def _layer(xin, W, b):
    C = xin.shape[1]
    Wa = W[:C] - W[C:]
    Wu = W[C:]
    if C < 8:
        pad = 8 - C
        xin = jnp.concatenate([xin, jnp.zeros((_N, pad), xin.dtype)], axis=1)
        Wa = jnp.concatenate([Wa, jnp.zeros((pad, _OUT), Wa.dtype)], axis=0)
        Wu = jnp.concatenate([Wu, jnp.zeros((pad, _OUT), Wu.dtype)], axis=0)
    u, a = _ua(xin, Wu, Wa, b.reshape(1, _OUT))
    xT2 = (xin + xin).T
    idxA = _edge_topk_half(xin, xT2, 0)
    idxB = _edge_topk_half(xin, xT2, 1)
    xnA = _sc_gather_relu_max(u, idxA.reshape(_HALF * _K), a[:_HALF])
    xnB = _sc_gather_relu_max(u, idxB.reshape(_HALF * _K), a[_HALF:])
    return jnp.concatenate([xnA, xnB], axis=0)


def kernel(x, W1, b1, W2, b2, W3, b3, Wl, bl, Wm1, bm1, Wm2, bm2, Wh, bh):
    x1 = _layer(x, W1, b1)
    x2 = _layer(x1, W2, b2)
    x3 = _layer(x2, W3, b3)
    return _head(x1, x2, x3, Wl, bl, Wm1, bm1, Wm2, bm2, Wh, bh)


# speedup vs baseline: 1.2060x; 1.0023x over previous
import functools

import jax
import jax.numpy as jnp
from jax import lax
from jax.experimental import pallas as pl
from jax.experimental.pallas import tpu as pltpu
from jax.experimental.pallas import tpu_sc as plsc

_N = 4096
_K = 32
_OUT = 64
_NEG = float(jnp.finfo(jnp.float32).min)
_HALF = _N // 2


def _dot(a, b):
    return lax.dot_general(a, b, (((1,), (0,)), ((), ())),
                           preferred_element_type=jnp.float32)


def _ua_body(x_ref, wu_ref, wa_ref, b_ref, u_ref, a_ref):
    xb = x_ref[...]
    u_ref[...] = _dot(xb, wu_ref[...])
    a_ref[...] = _dot(xb, wa_ref[...]) + b_ref[...]


def _ua(x, Wu, Wa, b2):
    C = x.shape[1]
    R = 2048
    return pl.pallas_call(
        _ua_body,
        grid=(_N // R,),
        in_specs=[
            pl.BlockSpec((R, C), lambda i: (i, 0)),
            pl.BlockSpec((C, _OUT), lambda i: (0, 0)),
            pl.BlockSpec((C, _OUT), lambda i: (0, 0)),
            pl.BlockSpec((1, _OUT), lambda i: (0, 0)),
        ],
        out_specs=[
            pl.BlockSpec((R, _OUT), lambda i: (i, 0)),
            pl.BlockSpec((R, _OUT), lambda i: (i, 0)),
        ],
        out_shape=[
            jax.ShapeDtypeStruct((_N, _OUT), jnp.float32),
            jax.ShapeDtypeStruct((_N, _OUT), jnp.float32),
        ],
    )(x, Wu, Wa, b2)


_TOPK_R = 256


def _topk_body(base_blk, x_ref, xT2_ref, idx_ref, s_ref):
    R = _TOPK_R
    xb = x_ref[...]
    xt2 = xT2_ref[...]
    xy2 = _dot(xb, xt2)
    sq = 0.25 * jnp.sum(xt2 * xt2, axis=0, keepdims=True)
    iota = lax.broadcasted_iota(jnp.int32, (R, _N), 1)
    iotaf = iota.astype(jnp.float32)
    lanek = lax.broadcasted_iota(jnp.int32, (R, _K), 1)
    blk = pl.program_id(0) + base_blk
    diag = lax.broadcasted_iota(jnp.int32, (R, 1), 0) + blk * R
    s_ref[...] = jnp.where(iota == diag, _NEG, xy2 - sq)

    def step(r, idx_acc):
        sw = s_ref[...]
        v = jnp.max(sw, axis=1, keepdims=True)
        jf = jnp.min(jnp.where(sw == v, iotaf, 1e9), axis=1, keepdims=True)
        jstar = jf.astype(jnp.int32)
        s_ref[...] = jnp.where(iota == jstar, _NEG, sw)
        return jnp.where(lanek == r, jstar, idx_acc)

    idx0 = jnp.where(lanek == 0, diag, jnp.zeros((R, _K), jnp.int32))
    idx_ref[...] = lax.fori_loop(1, _K, step, idx0)


def _edge_topk_half(x, xT2, half):
    C = x.shape[1]
    R = _TOPK_R
    G = _HALF // R
    base_blk = half * G
    return pl.pallas_call(
        functools.partial(_topk_body, base_blk),
        grid=(G,),
        in_specs=[
            pl.BlockSpec((R, C), lambda i: (i + base_blk, 0)),
            pl.BlockSpec((C, _N), lambda i: (0, 0)),
        ],
        out_specs=pl.BlockSpec((R, _K), lambda i: (i, 0)),
        out_shape=jax.ShapeDtypeStruct((_HALF, _K), jnp.int32),
        scratch_shapes=[pltpu.VMEM((R, _N), jnp.float32)],
    )(x, xT2)


_NW = 32
_ROWS_W = _HALF // _NW
_CH = 4
_NCH = _ROWS_W // _CH


def _sc_gather_relu_max(u, idx_flat, a_half):
    mesh = plsc.VectorSubcoreMesh(core_axis_name="c", subcore_axis_name="s")

    @functools.partial(
        pl.kernel,
        out_type=jax.ShapeDtypeStruct((_HALF, _OUT), jnp.float32),
        mesh=mesh,
        compiler_params=pltpu.CompilerParams(use_tc_tiling_on_sc=False),
        scratch_types=[
            pltpu.VMEM((_CH * _K,), jnp.int32),
            pltpu.VMEM((_CH * _K,), jnp.int32),
            pltpu.VMEM((_CH * _K, _OUT), jnp.float32),
            pltpu.VMEM((_CH * _K, _OUT), jnp.float32),
            pltpu.VMEM((_ROWS_W, _OUT), jnp.float32),
            pltpu.VMEM((_ROWS_W, _OUT), jnp.float32),
            pltpu.SemaphoreType.DMA,
            pltpu.SemaphoreType.DMA,
            pltpu.SemaphoreType.DMA,
        ],
    )
    def k(u_hbm, idx_hbm, a_hbm, out_hbm, idx0, idx1, rows0, rows1,
          a_v, o_v, sem0, sem1, sema):
        wid = lax.axis_index("s") * 2 + lax.axis_index("c")
        base = wid * _ROWS_W

        def fire(ci, idxbuf, rowsbuf, sem):
            pltpu.sync_copy(
                idx_hbm.at[pl.ds((base + ci * _CH) * _K, _CH * _K)], idxbuf)
            pltpu.async_copy(u_hbm.at[idxbuf], rowsbuf, sem)

        def wait(rowsbuf, sem):
            pltpu.make_async_copy(u_hbm.at[pl.ds(0, _CH * _K), :],
                                  rowsbuf, sem).wait()

        def compute(ci, rowsbuf):
            def row_do(r, _):
                def jstep(j, accs):
                    return tuple(
                        jnp.maximum(acc, rowsbuf[r * _K + j, pl.ds(c * 16, 16)])
                        for c, acc in enumerate(accs))
                accs = tuple(rowsbuf[r * _K, pl.ds(c * 16, 16)]
                             for c in range(_OUT // 16))
                accs = lax.fori_loop(1, _K, jstep, accs)
                orow = ci * _CH + r
                for c in range(_OUT // 16):
                    o_v[orow, pl.ds(c * 16, 16)] = jnp.maximum(
                        a_v[orow, pl.ds(c * 16, 16)] + accs[c], 0.0)
                return 0
            lax.fori_loop(0, _CH, row_do, 0)

        pltpu.async_copy(a_hbm.at[pl.ds(base, _ROWS_W), :], a_v, sema)
        fire(0, idx0, rows0, sem0)
        pltpu.make_async_copy(a_hbm.at[pl.ds(base, _ROWS_W), :],
                              a_v, sema).wait()

        def body(g, carry):
            ci0 = 2 * g
            fire(ci0 + 1, idx1, rows1, sem1)
            wait(rows0, sem0)
            compute(ci0, rows0)

            @pl.when(ci0 + 2 < _NCH)
            def _():
                fire(ci0 + 2, idx0, rows0, sem0)

            wait(rows1, sem1)
            compute(ci0 + 1, rows1)
            return carry

        lax.fori_loop(0, _NCH // 2, body, 0)
        pltpu.sync_copy(o_v, out_hbm.at[pl.ds(base, _ROWS_W), :])

    return k(u, idx_flat, a_half)


_HEAD_R = 1024


def _head_body(x1_ref, x2_ref, x3_ref, wl_ref, bl_ref, wm1_ref, bm1_ref,
               wm2_ref, bm2_ref, wh_ref, bh_ref, o_ref):
    wl = wl_ref[...]
    h = (_dot(x1_ref[...], wl[0:_OUT]) + _dot(x2_ref[...], wl[_OUT:2 * _OUT])
         + _dot(x3_ref[...], wl[2 * _OUT:3 * _OUT]) + bl_ref[...])
    h = jnp.maximum(h, 0.0)
    h = jnp.maximum(_dot(h, wm1_ref[...]) + bm1_ref[...], 0.0)
    h = jnp.maximum(_dot(h, wm2_ref[...]) + bm2_ref[...], 0.0)
    o = _dot(h, wh_ref[...]) + bh_ref[...]
    shifted = o - jnp.max(o, axis=1, keepdims=True)
    o_ref[...] = shifted - jnp.log(
        jnp.sum(jnp.exp(shifted), axis=1, keepdims=True))


def _head(x1, x2, x3, Wl, bl, Wm1, bm1, Wm2, bm2, Wh, bh):
    R = _HEAD_R
    G = _N // R
    ncls = Wh.shape[1]
    full = lambda shp: pl.BlockSpec(shp, lambda i: tuple(0 for _ in shp))
    row = lambda shp: pl.BlockSpec(shp, lambda i: (i,) + (0,) * (len(shp) - 1))
    return pl.pallas_call(
        _head_body,
        grid=(G,),
        in_specs=[
            row((R, _OUT)), row((R, _OUT)), row((R, _OUT)),
            full(Wl.shape), full((1, bl.shape[0])),
            full(Wm1.shape), full((1, bm1.shape[0])),
            full(Wm2.shape), full((1, bm2.shape[0])),
            full(Wh.shape), full((1, bh.shape[0])),
        ],
        out_specs=row((R, ncls)),
        out_shape=jax.ShapeDtypeStruct((_N, ncls), jnp.float32),
    )(x1, x2, x3, Wl, bl.reshape(1, -1), Wm1, bm1.reshape(1, -1),
      Wm2, bm2.reshape(1, -1), Wh, bh.reshape(1, -1))


def _layer(xin, W, b):
    C = xin.shape[1]
    Wa = W[:C] - W[C:]
    Wu = W[C:]
    if C < 8:
        pad = 8 - C
        xin = jnp.concatenate([xin, jnp.zeros((_N, pad), xin.dtype)], axis=1)
        Wa = jnp.concatenate([Wa, jnp.zeros((pad, _OUT), Wa.dtype)], axis=0)
        Wu = jnp.concatenate([Wu, jnp.zeros((pad, _OUT), Wu.dtype)], axis=0)
    u, a = _ua(xin, Wu, Wa, b.reshape(1, _OUT))
    xT2 = (xin + xin).T
    idxA = _edge_topk_half(xin, xT2, 0)
    idxB = _edge_topk_half(xin, xT2, 1)
    xnA = _sc_gather_relu_max(u, idxA.reshape(_HALF * _K), a[:_HALF])
    xnB = _sc_gather_relu_max(u, idxB.reshape(_HALF * _K), a[_HALF:])
    return jnp.concatenate([xnA, xnB], axis=0)


def kernel(x, W1, b1, W2, b2, W3, b3, Wl, bl, Wm1, bm1, Wm2, bm2, Wh, bh):
    x1 = _layer(x, W1, b1)
    x2 = _layer(x1, W2, b2)
    x3 = _layer(x2, W3, b3)
    return _head(x1, x2, x3, Wl, bl, Wm1, bm1, Wm2, bm2, Wh, bh)
